# baseline (speedup 1.0000x reference)
import functools

import jax
import jax.numpy as jnp
from jax.experimental import pallas as pl
from jax.experimental.pallas import tpu as pltpu

B, H, L, E = 32, 16, 192, 3
BH = B * H
TOP_P = 0.5
NOISE_EPS = 0.01
EPS_C = 1e-10
BLK = 16
GRID = BH // BLK
BLKC = 8
GRIDC = BH // BLKC

def _gen_eps_t():
    return jnp.transpose(
        jax.random.normal(jax.random.key(42), (BH, L, E), dtype=jnp.float32),
        (0, 2, 1))


try:
    _EPS_CONST = jax.device_get(_gen_eps_t())
except Exception:
    _EPS_CONST = None


def _eps_t():
    return _gen_eps_t() if _EPS_CONST is None else _EPS_CONST


def _gate_body(x_ref, eps_ref, w_ref, km_ref, loss_ref, acc_ref):
    step = pl.program_id(0)

    @pl.when(step == 0)
    def _init():
        acc_ref[...] = jnp.zeros_like(acc_ref)

    s0_tot = jnp.zeros((1, L), jnp.float32)
    s1_tot = jnp.zeros((1, L), jnp.float32)
    ent_tot = jnp.zeros((1, L), jnp.float32)
    zero = jnp.zeros((1, L), jnp.float32)
    one = jnp.ones((1, L), jnp.float32)

    for i in range(BLK):
        a = x_ref[i]
        res = jnp.dot(a, w_ref[...], preferred_element_type=jnp.float32)
        t = jnp.transpose(res)
        ee = eps_ref[i]
        n0 = t[0:1, :] + ee[0:1, :] * (jax.nn.softplus(t[3:4, :]) + NOISE_EPS)
        n1 = t[1:2, :] + ee[1:2, :] * (jax.nn.softplus(t[4:5, :]) + NOISE_EPS)
        n2 = t[2:3, :] + ee[2:3, :] * (jax.nn.softplus(t[5:6, :]) + NOISE_EPS)
        mx = jnp.maximum(jnp.maximum(n0, n1), n2)
        d0 = n0 - mx
        d1 = n1 - mx
        d2 = n2 - mx
        x0 = jnp.exp(d0)
        x1 = jnp.exp(d1)
        x2 = jnp.exp(d2)
        z = x0 + x1 + x2
        rz = 1.0 / z
        p0 = x0 * rz
        p1 = x1 * rz
        p2 = x2 * rz
        ent_tot += p0 * d0 + p1 * d1 + p2 * d2 - jnp.log(z)
        a10 = p1 > p0
        a20 = p2 > p0
        a01 = p0 >= p1
        a21 = p2 > p1
        a02 = p0 >= p2
        a12 = p1 >= p2
        pmax = jnp.maximum(jnp.maximum(p0, p1), p2)
        pmin = jnp.minimum(jnp.minimum(p0, p1), p2)
        pmed = (p0 + p1 + p2) - pmax - pmin
        phi = pmax <= TOP_P
        s0_tot += pmax
        s1_tot += jnp.where(phi, pmed, zero)
        k0 = (a01 & a02) | ((a01 ^ a02) & phi)
        k1 = (a10 & a12) | ((a10 ^ a12) & phi)
        k2 = (a20 & a21) | ((a20 ^ a21) & phi)
        km_ref[i, 0:1, :] = jnp.where(k0, one, zero)
        km_ref[i, 1:2, :] = jnp.where(k1, one, zero)
        km_ref[i, 2:3, :] = jnp.where(k2, one, zero)

    acc_ref[0:1, :] += s0_tot
    acc_ref[1:2, :] += s1_tot
    acc_ref[2:3, :] += ent_tot

    @pl.when(step == GRID - 1)
    def _finalize():
        s0 = acc_ref[0:1, :]
        s1 = acc_ref[1:2, :]
        n = float(L * E)
        tot = jnp.sum(s0) + jnp.sum(s1)
        sq = jnp.sum(s0 * s0) + jnp.sum(s1 * s1)
        mean = tot / n
        var = (sq - n * mean * mean) / (n - 1.0)
        loss_imp = var / (mean * mean + EPS_C)
        loss_dyn = -jnp.sum(acc_ref[2:3, :]) / float(BH * E)
        loss_ref[...] = jnp.reshape(loss_imp + 0.1 * loss_dyn, (1, 1))


def _combine_body(km_ref, m_ref, out_ref):
    m0 = m_ref[0]
    m1 = m_ref[1]
    m2 = m_ref[2]
    rows = jax.lax.broadcasted_iota(jnp.int32, (L, L), 0)
    cols = jax.lax.broadcasted_iota(jnp.int32, (L, L), 1)
    eye = jnp.where(rows == cols, 1.0, 0.0)
    for i in range(BLKC):
        kt = jnp.transpose(km_ref[i])
        out_ref[i] = (kt[:, 0:1] * m0 + kt[:, 1:2] * m1 + kt[:, 2:3] * m2
                      + eye)


@functools.partial(jax.jit, static_argnames=())
def kernel(x, masks, W_gate, W_noise):
    xf = x.reshape(BH, L, L)
    eps_t = _eps_t()
    w = jnp.concatenate(
        [W_gate, W_noise, jnp.zeros((2, L), jnp.float32)], axis=0).T
    masks_t = jnp.transpose(masks, (1, 0, 2))

    km, loss = pl.pallas_call(
        _gate_body,
        grid=(GRID,),
        in_specs=[
            pl.BlockSpec((BLK, L, L), lambda i: (i, 0, 0)),
            pl.BlockSpec((BLK, E, L), lambda i: (i, 0, 0)),
            pl.BlockSpec((L, 8), lambda i: (0, 0)),
        ],
        out_specs=[
            pl.BlockSpec((BLK, 8, L), lambda i: (i, 0, 0)),
            pl.BlockSpec((1, 1), lambda i: (0, 0)),
        ],
        out_shape=[
            jax.ShapeDtypeStruct((BH, 8, L), jnp.float32),
            jax.ShapeDtypeStruct((1, 1), jnp.float32),
        ],
        scratch_shapes=[
            pltpu.VMEM((8, L), jnp.float32),
        ],
        compiler_params=pltpu.CompilerParams(
            dimension_semantics=("arbitrary",),
        ),
    )(xf, eps_t, w)

    out = pl.pallas_call(
        _combine_body,
        grid=(GRIDC,),
        in_specs=[
            pl.BlockSpec((BLKC, 8, L), lambda i: (i, 0, 0)),
            pl.BlockSpec((E, L, L), lambda i: (0, 0, 0)),
        ],
        out_specs=pl.BlockSpec((BLKC, L, L), lambda i: (i, 0, 0)),
        out_shape=jax.ShapeDtypeStruct((BH, L, L), jnp.float32),
        compiler_params=pltpu.CompilerParams(
            dimension_semantics=("arbitrary",),
        ),
    )(km, masks_t)
    return out.reshape(B, H, L, L), loss[0, 0]


# speedup vs baseline: 1.0898x; 1.0898x over previous
import functools

import jax
import jax.numpy as jnp
from jax.experimental import pallas as pl
from jax.experimental.pallas import tpu as pltpu

B, H, L, E = 32, 16, 192, 3
BH = B * H
TOP_P = 0.5
NOISE_EPS = 0.01
EPS_C = 1e-10
BLK = 16
GRID = BH // BLK


def _moe_body(x_ref, eps_ref, w_ref, m_ref, out_ref, loss_ref, acc_ref):
    step = pl.program_id(0)

    @pl.when(step == 0)
    def _init():
        acc_ref[...] = jnp.zeros_like(acc_ref)

    m0 = m_ref[0]
    m1 = m_ref[1]
    m2 = m_ref[2]
    rows = jax.lax.broadcasted_iota(jnp.int32, (L, L), 0)
    cols = jax.lax.broadcasted_iota(jnp.int32, (L, L), 1)
    eye = jnp.where(rows == cols, 1.0, 0.0)

    s0_tot = jnp.zeros((1, L), jnp.float32)
    s1_tot = jnp.zeros((1, L), jnp.float32)
    ent_tot = jnp.zeros((1, L), jnp.float32)
    zero = jnp.zeros((1, L), jnp.float32)
    one = jnp.ones((1, L), jnp.float32)

    for i in range(BLK):
        a = x_ref[i]
        res = jnp.dot(a, w_ref[...], preferred_element_type=jnp.float32)
        t = jnp.transpose(res)
        ee = eps_ref[i]
        n0 = t[0:1, :] + ee[0:1, :] * (jax.nn.softplus(t[3:4, :]) + NOISE_EPS)
        n1 = t[1:2, :] + ee[1:2, :] * (jax.nn.softplus(t[4:5, :]) + NOISE_EPS)
        n2 = t[2:3, :] + ee[2:3, :] * (jax.nn.softplus(t[5:6, :]) + NOISE_EPS)
        mx = jnp.maximum(jnp.maximum(n0, n1), n2)
        d0 = n0 - mx
        d1 = n1 - mx
        d2 = n2 - mx
        x0 = jnp.exp(d0)
        x1 = jnp.exp(d1)
        x2 = jnp.exp(d2)
        z = x0 + x1 + x2
        rz = 1.0 / z
        p0 = x0 * rz
        p1 = x1 * rz
        p2 = x2 * rz
        ent_tot += p0 * d0 + p1 * d1 + p2 * d2 - jnp.log(z)
        a10 = p1 > p0
        a20 = p2 > p0
        a01 = p0 >= p1
        a21 = p2 > p1
        a02 = p0 >= p2
        a12 = p1 >= p2
        pmax = jnp.maximum(jnp.maximum(p0, p1), p2)
        pmin = jnp.minimum(jnp.minimum(p0, p1), p2)
        pmed = (p0 + p1 + p2) - pmax - pmin
        phi = pmax <= TOP_P
        s0_tot += pmax
        s1_tot += jnp.where(phi, pmed, zero)
        k0 = (a01 & a02) | ((a01 ^ a02) & phi)
        k1 = (a10 & a12) | ((a10 ^ a12) & phi)
        k2 = (a20 & a21) | ((a20 ^ a21) & phi)
        kmat = jnp.concatenate(
            [jnp.where(k0, one, zero), jnp.where(k1, one, zero),
             jnp.where(k2, one, zero)], axis=0)
        kt = jnp.transpose(kmat)
        out_ref[i] = (kt[:, 0:1] * m0 + kt[:, 1:2] * m1 + kt[:, 2:3] * m2
                      + eye)

    acc_ref[0:1, :] += s0_tot
    acc_ref[1:2, :] += s1_tot
    acc_ref[2:3, :] += ent_tot

    @pl.when(step == GRID - 1)
    def _finalize():
        s0 = acc_ref[0:1, :]
        s1 = acc_ref[1:2, :]
        n = float(L * E)
        tot = jnp.sum(s0) + jnp.sum(s1)
        sq = jnp.sum(s0 * s0) + jnp.sum(s1 * s1)
        mean = tot / n
        var = (sq - n * mean * mean) / (n - 1.0)
        loss_imp = var / (mean * mean + EPS_C)
        loss_dyn = -jnp.sum(acc_ref[2:3, :]) / float(BH * E)
        loss_ref[...] = jnp.reshape(loss_imp + 0.1 * loss_dyn, (1, 1))


def _gen_eps_t():
    return jnp.transpose(
        jax.random.normal(jax.random.key(42), (BH, L, E), dtype=jnp.float32),
        (0, 2, 1))


try:
    _EPS_CONST = jax.device_get(_gen_eps_t())
except Exception:
    _EPS_CONST = None


@functools.partial(jax.jit, static_argnames=())
def kernel(x, masks, W_gate, W_noise):
    xf = x.reshape(BH, L, L)
    eps_t = _gen_eps_t() if _EPS_CONST is None else _EPS_CONST
    w = jnp.concatenate(
        [W_gate, W_noise, jnp.zeros((2, L), jnp.float32)], axis=0).T
    masks_t = jnp.transpose(masks, (1, 0, 2))
    out, loss = pl.pallas_call(
        _moe_body,
        grid=(GRID,),
        in_specs=[
            pl.BlockSpec((BLK, L, L), lambda i: (i, 0, 0)),
            pl.BlockSpec((BLK, E, L), lambda i: (i, 0, 0)),
            pl.BlockSpec((L, 8), lambda i: (0, 0)),
            pl.BlockSpec((E, L, L), lambda i: (0, 0, 0)),
        ],
        out_specs=[
            pl.BlockSpec((BLK, L, L), lambda i: (i, 0, 0)),
            pl.BlockSpec((1, 1), lambda i: (0, 0)),
        ],
        out_shape=[
            jax.ShapeDtypeStruct((BH, L, L), jnp.float32),
            jax.ShapeDtypeStruct((1, 1), jnp.float32),
        ],
        scratch_shapes=[
            pltpu.VMEM((8, L), jnp.float32),
        ],
        compiler_params=pltpu.CompilerParams(
            dimension_semantics=("arbitrary",),
        ),
    )(xf, eps_t, w, masks_t)
    return out.reshape(B, H, L, L), loss[0, 0]
